# initial kernel scaffold (unmeasured)
import functools

import numpy as np

import jax
import jax.numpy as jnp
from jax import lax
from jax.experimental import pallas as pl
from jax.experimental.pallas import tpu as pltpu

N_DEV = 16
B = 2
SQ = 256
D = 768
NH = 4
DH = 64
HL = NH * DH

_HIGH = lax.Precision.HIGHEST


def _rope_tables():
    inv = 1.0 / (10000.0 ** (np.arange(0, DH, 2) / DH))
    ang = np.arange(SQ)[:, None] * inv[None, :]
    cos = np.repeat(np.cos(ang), 2, axis=-1).astype(np.float32)
    sin = np.repeat(np.sin(ang), 2, axis=-1).astype(np.float32)
    cos_t = np.tile(cos, (B, NH))
    sin_t = np.tile(sin, (B, NH))
    p = np.zeros((DH, DH), dtype=np.float32)
    for k in range(DH // 2):
        p[2 * k + 1, 2 * k] = -1.0
        p[2 * k, 2 * k + 1] = 1.0
    p4 = np.kron(np.eye(NH, dtype=np.float32), p)
    return cos_t, sin_t, p4


def kernel(x, Wq, Wk, Wv, Wo):
    cos_t, sin_t, p4 = _rope_tables()

    w_stacked = jnp.concatenate([Wq, Wk, Wv, Wo.T], axis=0)
    x2 = x.reshape(B * SQ, D)

    def body(x_ref, w_ref, p_ref, cos_ref, sin_ref, out_ref,
             gbuf, ctx_buf, send_sems, recv_sems):
        me = lax.axis_index("i")
        right = lax.rem(me + 1, N_DEV)
        left = lax.rem(me + N_DEV - 1, N_DEV)

        barrier_sem = pltpu.get_barrier_semaphore()
        for nbr in (left, right):
            pl.semaphore_signal(
                barrier_sem, inc=1,
                device_id=(nbr,), device_id_type=pl.DeviceIdType.MESH,
            )
        pl.semaphore_wait(barrier_sem, 2)

        gbuf[0] = w_ref[...]
        out_ref[...] = jnp.zeros_like(out_ref)

        def compute_chunk(s):
            blk = gbuf[s]
            wq = blk[0 * D:1 * D]
            wk = blk[1 * D:2 * D]
            wv = blk[2 * D:3 * D]
            woT = blk[3 * D:4 * D]
            xv = x_ref[...]
            q = jnp.dot(xv, wq, precision=_HIGH)
            k = jnp.dot(xv, wk, precision=_HIGH)
            v = jnp.dot(xv, wv, precision=_HIGH)
            cs = cos_ref[...]
            sn = sin_ref[...]
            pm = p_ref[...]
            qr = q * cs + jnp.dot(q, pm, precision=_HIGH) * sn
            kr = k * cs + jnp.dot(k, pm, precision=_HIGH) * sn
            for b in range(B):
                rs = slice(b * SQ, (b + 1) * SQ)
                for hh in range(NH):
                    cs_h = slice(hh * DH, (hh + 1) * DH)
                    qh = qr[rs, cs_h]
                    kh = kr[rs, cs_h]
                    vh = v[rs, cs_h]
                    sc = lax.dot_general(
                        qh, kh, (((1,), (1,)), ((), ())), precision=_HIGH,
                    ) * 0.125
                    m = jnp.max(sc, axis=1, keepdims=True)
                    e = jnp.exp(sc - m)
                    sm = e / jnp.sum(e, axis=1, keepdims=True)
                    ctx_buf[rs, cs_h] = jnp.dot(sm, vh, precision=_HIGH)
            out_ref[...] += lax.dot_general(
                ctx_buf[...], woT, (((1,), (1,)), ((), ())), precision=_HIGH,
            )

        for h in range(1, N_DEV):
            rdma = pltpu.make_async_remote_copy(
                src_ref=gbuf.at[h - 1],
                dst_ref=gbuf.at[h],
                send_sem=send_sems.at[h],
                recv_sem=recv_sems.at[h],
                device_id=(right,),
                device_id_type=pl.DeviceIdType.MESH,
            )
            rdma.start()
            compute_chunk(h - 1)
            rdma.wait()
        compute_chunk(N_DEV - 1)

    out = pl.pallas_call(
        body,
        out_shape=jax.ShapeDtypeStruct((B * SQ, D), jnp.float32),
        in_specs=[
            pl.BlockSpec(memory_space=pltpu.VMEM),
            pl.BlockSpec(memory_space=pltpu.VMEM),
            pl.BlockSpec(memory_space=pltpu.VMEM),
            pl.BlockSpec(memory_space=pltpu.VMEM),
            pl.BlockSpec(memory_space=pltpu.VMEM),
        ],
        out_specs=pl.BlockSpec(memory_space=pltpu.VMEM),
        scratch_shapes=[
            pltpu.VMEM((N_DEV, 4 * D, HL), jnp.float32),
            pltpu.VMEM((B * SQ, HL), jnp.float32),
            pltpu.SemaphoreType.DMA((N_DEV,)),
            pltpu.SemaphoreType.DMA((N_DEV,)),
        ],
        compiler_params=pltpu.CompilerParams(collective_id=0),
    )(x2, jnp.asarray(w_stacked), jnp.asarray(p4),
      jnp.asarray(cos_t), jnp.asarray(sin_t))

    return out.reshape(B, SQ, D)


# baseline (device time: 559618 ns/iter reference)
import functools

import numpy as np

import jax
import jax.numpy as jnp
from jax import lax
from jax.experimental import pallas as pl
from jax.experimental.pallas import tpu as pltpu

N_DEV = 16
B = 2
SQ = 256
D = 768
NH = 4
DH = 64
HL = NH * DH

_HIGH = lax.Precision.HIGHEST


def _rope_tables():
    inv = 1.0 / (10000.0 ** (np.arange(0, DH, 2) / DH))
    ang = np.arange(SQ)[:, None] * inv[None, :]
    cos = np.repeat(np.cos(ang), 2, axis=-1).astype(np.float32)
    sin = np.repeat(np.sin(ang), 2, axis=-1).astype(np.float32)
    cos_t = np.tile(cos, (B, NH))
    sin_t = np.tile(sin, (B, NH))
    p = np.zeros((DH, DH), dtype=np.float32)
    for k in range(DH // 2):
        p[2 * k + 1, 2 * k] = -1.0
        p[2 * k, 2 * k + 1] = 1.0
    p4 = np.kron(np.eye(NH, dtype=np.float32), p)
    return cos_t, sin_t, p4


def kernel(x, Wq, Wk, Wv, Wo):
    cos_t, sin_t, p4 = _rope_tables()

    w_stacked = jnp.concatenate([Wq, Wk, Wv, Wo.T], axis=0)
    x2 = x.reshape(B * SQ, D)

    def body(x_ref, w_ref, p_ref, cos_ref, sin_ref, out_ref,
             gbuf, ctx_buf, send_sems, recv_sems):
        me = lax.axis_index("i")
        right = lax.rem(me + 1, N_DEV)
        left = lax.rem(me + N_DEV - 1, N_DEV)

        barrier_sem = pltpu.get_barrier_semaphore()
        for nbr in (left, right):
            pl.semaphore_signal(
                barrier_sem, inc=1,
                device_id=(nbr,), device_id_type=pl.DeviceIdType.MESH,
            )
        pl.semaphore_wait(barrier_sem, 2)

        out_ref[...] = jnp.zeros_like(out_ref)

        def compute_chunk(s):
            blk = w_ref[...] if s == 0 else gbuf[s - 1]
            wq = blk[0 * D:1 * D]
            wk = blk[1 * D:2 * D]
            wv = blk[2 * D:3 * D]
            woT = blk[3 * D:4 * D]
            xv = x_ref[...]
            q = jnp.dot(xv, wq, precision=_HIGH)
            k = jnp.dot(xv, wk, precision=_HIGH)
            v = jnp.dot(xv, wv, precision=_HIGH)
            cs = cos_ref[...]
            sn = sin_ref[...]
            pm = p_ref[...]
            qr = q * cs + jnp.dot(q, pm, precision=_HIGH) * sn
            kr = k * cs + jnp.dot(k, pm, precision=_HIGH) * sn
            for b in range(B):
                rs = slice(b * SQ, (b + 1) * SQ)
                for hh in range(NH):
                    cs_h = slice(hh * DH, (hh + 1) * DH)
                    qh = qr[rs, cs_h]
                    kh = kr[rs, cs_h]
                    vh = v[rs, cs_h]
                    sc = lax.dot_general(
                        qh, kh, (((1,), (1,)), ((), ())), precision=_HIGH,
                    ) * 0.125
                    m = jnp.max(sc, axis=1, keepdims=True)
                    e = jnp.exp(sc - m)
                    sm = e / jnp.sum(e, axis=1, keepdims=True)
                    ctx_buf[rs, cs_h] = jnp.dot(sm, vh, precision=_HIGH)
            out_ref[...] += lax.dot_general(
                ctx_buf[...], woT, (((1,), (1,)), ((), ())), precision=_HIGH,
            )

        for h in range(1, N_DEV):
            rdma = pltpu.make_async_remote_copy(
                src_ref=w_ref if h == 1 else gbuf.at[h - 2],
                dst_ref=gbuf.at[h - 1],
                send_sem=send_sems.at[h],
                recv_sem=recv_sems.at[h],
                device_id=(right,),
                device_id_type=pl.DeviceIdType.MESH,
            )
            rdma.start()
            compute_chunk(h - 1)
            rdma.wait()
        compute_chunk(N_DEV - 1)

    out = pl.pallas_call(
        body,
        out_shape=jax.ShapeDtypeStruct((B * SQ, D), jnp.float32),
        in_specs=[
            pl.BlockSpec(memory_space=pltpu.VMEM),
            pl.BlockSpec(memory_space=pltpu.VMEM),
            pl.BlockSpec(memory_space=pltpu.VMEM),
            pl.BlockSpec(memory_space=pltpu.VMEM),
            pl.BlockSpec(memory_space=pltpu.VMEM),
        ],
        out_specs=pl.BlockSpec(memory_space=pltpu.VMEM),
        scratch_shapes=[
            pltpu.VMEM((N_DEV - 1, 4 * D, HL), jnp.float32),
            pltpu.VMEM((B * SQ, HL), jnp.float32),
            pltpu.SemaphoreType.DMA((N_DEV,)),
            pltpu.SemaphoreType.DMA((N_DEV,)),
        ],
        compiler_params=pltpu.CompilerParams(
            collective_id=0,
            vmem_limit_bytes=64 * 1024 * 1024,
        ),
    )(x2, jnp.asarray(w_stacked), jnp.asarray(p4),
      jnp.asarray(cos_t), jnp.asarray(sin_t))

    return out.reshape(B, SQ, D)


# device time: 179824 ns/iter; 3.1120x vs baseline; 3.1120x over previous
import numpy as np

import jax
import jax.numpy as jnp
from jax import lax
from jax.experimental import pallas as pl
from jax.experimental.pallas import tpu as pltpu

N_DEV = 16
NR = 8
NL = 7
B = 2
SQ = 256
D = 768
NH = 4
DH = 64
HL = NH * DH

F32 = jnp.float32
BF16 = jnp.bfloat16


def _rope_tables():
    inv = 1.0 / (10000.0 ** (np.arange(0, DH, 2) / DH))
    ang = np.arange(SQ)[:, None] * inv[None, :]
    cos = np.repeat(np.cos(ang), 2, axis=-1).astype(np.float32)
    sin = np.repeat(np.sin(ang), 2, axis=-1).astype(np.float32)
    cos_t = np.tile(cos, (B, NH))
    sin_t = np.tile(sin, (B, NH))
    p = np.zeros((DH, DH), dtype=np.float32)
    for k in range(DH // 2):
        p[2 * k + 1, 2 * k] = -1.0
        p[2 * k, 2 * k + 1] = 1.0
    p4 = np.kron(np.eye(NH, dtype=np.float32), p)
    return cos_t, sin_t, p4


def kernel(x, Wq, Wk, Wv, Wo):
    cos_t, sin_t, p4 = _rope_tables()

    w_stacked = jnp.concatenate([Wq, Wk, Wv, Wo.T], axis=0).astype(BF16)
    x2 = x.reshape(B * SQ, D).astype(BF16)

    def body(x_ref, w_ref, p_ref, cos_ref, sin_ref, out_ref,
             gbuf_r, gbuf_l, ctx_buf,
             send_r, recv_r, send_l, recv_l):
        me = lax.axis_index("i")
        right = lax.rem(me + 1, N_DEV)
        left = lax.rem(me + N_DEV - 1, N_DEV)

        barrier_sem = pltpu.get_barrier_semaphore()
        for nbr in (left, right):
            pl.semaphore_signal(
                barrier_sem, inc=1,
                device_id=(nbr,), device_id_type=pl.DeviceIdType.MESH,
            )
        pl.semaphore_wait(barrier_sem, 2)

        out_ref[...] = jnp.zeros_like(out_ref)
        xv = x_ref[...]
        cs = cos_ref[...]
        sn = sin_ref[...]
        pm = p_ref[...]

        def compute_chunk(blk):
            wq = blk[0 * D:1 * D]
            wk = blk[1 * D:2 * D]
            wv = blk[2 * D:3 * D]
            woT = blk[3 * D:4 * D]
            q = jnp.dot(xv, wq, preferred_element_type=F32)
            k = jnp.dot(xv, wk, preferred_element_type=F32)
            v = jnp.dot(xv, wv, preferred_element_type=F32)
            qr = q * cs + jnp.dot(q.astype(BF16), pm,
                                  preferred_element_type=F32) * sn
            kr = k * cs + jnp.dot(k.astype(BF16), pm,
                                  preferred_element_type=F32) * sn
            qr = qr.astype(BF16)
            kr = kr.astype(BF16)
            vb = v.astype(BF16)
            for b in range(B):
                rs = slice(b * SQ, (b + 1) * SQ)
                for hh in range(NH):
                    cs_h = slice(hh * DH, (hh + 1) * DH)
                    qh = qr[rs, cs_h]
                    kh = kr[rs, cs_h]
                    vh = vb[rs, cs_h]
                    sc = lax.dot_general(
                        qh, kh, (((1,), (1,)), ((), ())),
                        preferred_element_type=F32,
                    ) * 0.125
                    m = jnp.max(sc, axis=1, keepdims=True)
                    e = jnp.exp(sc - m)
                    sm = (e / jnp.sum(e, axis=1, keepdims=True)).astype(BF16)
                    ctx_buf[rs, cs_h] = jnp.dot(
                        sm, vh, preferred_element_type=F32).astype(BF16)
            out_ref[...] += lax.dot_general(
                ctx_buf[...], woT, (((1,), (1,)), ((), ())),
                preferred_element_type=F32,
            )

        for h in range(1, NR + 1):
            rd_r = pltpu.make_async_remote_copy(
                src_ref=w_ref if h == 1 else gbuf_r.at[h - 2],
                dst_ref=gbuf_r.at[h - 1],
                send_sem=send_r.at[h - 1],
                recv_sem=recv_r.at[h - 1],
                device_id=(right,),
                device_id_type=pl.DeviceIdType.MESH,
            )
            rd_r.start()
            rd_l = None
            if h <= NL:
                rd_l = pltpu.make_async_remote_copy(
                    src_ref=w_ref if h == 1 else gbuf_l.at[h - 2],
                    dst_ref=gbuf_l.at[h - 1],
                    send_sem=send_l.at[h - 1],
                    recv_sem=recv_l.at[h - 1],
                    device_id=(left,),
                    device_id_type=pl.DeviceIdType.MESH,
                )
                rd_l.start()
            if h == 1:
                compute_chunk(w_ref[...])
            else:
                compute_chunk(gbuf_r[h - 2])
                compute_chunk(gbuf_l[h - 2])
            rd_r.wait()
            if rd_l is not None:
                rd_l.wait()
        compute_chunk(gbuf_r[NR - 1])

    out = pl.pallas_call(
        body,
        out_shape=jax.ShapeDtypeStruct((B * SQ, D), F32),
        in_specs=[
            pl.BlockSpec(memory_space=pltpu.VMEM),
            pl.BlockSpec(memory_space=pltpu.VMEM),
            pl.BlockSpec(memory_space=pltpu.VMEM),
            pl.BlockSpec(memory_space=pltpu.VMEM),
            pl.BlockSpec(memory_space=pltpu.VMEM),
        ],
        out_specs=pl.BlockSpec(memory_space=pltpu.VMEM),
        scratch_shapes=[
            pltpu.VMEM((NR, 4 * D, HL), BF16),
            pltpu.VMEM((NL, 4 * D, HL), BF16),
            pltpu.VMEM((B * SQ, HL), BF16),
            pltpu.SemaphoreType.DMA((NR,)),
            pltpu.SemaphoreType.DMA((NR,)),
            pltpu.SemaphoreType.DMA((NL,)),
            pltpu.SemaphoreType.DMA((NL,)),
        ],
        compiler_params=pltpu.CompilerParams(
            collective_id=0,
            vmem_limit_bytes=64 * 1024 * 1024,
        ),
    )(x2, w_stacked, jnp.asarray(p4, dtype=BF16),
      jnp.asarray(cos_t), jnp.asarray(sin_t))

    return out.reshape(B, SQ, D)


# device time: 157466 ns/iter; 3.5539x vs baseline; 1.1420x over previous
import numpy as np

import jax
import jax.numpy as jnp
from jax import lax
from jax.experimental import pallas as pl
from jax.experimental.pallas import tpu as pltpu

N_DEV = 16
NR = 8
NL = 7
B = 2
SQ = 256
D = 768
NH = 4
DH = 64
HL = NH * DH

F32 = jnp.float32
BF16 = jnp.bfloat16


def _rope_tables():
    inv = 1.0 / (10000.0 ** (np.arange(0, DH, 2) / DH))
    ang = np.arange(SQ)[:, None] * inv[None, :]
    cos = np.repeat(np.cos(ang), 2, axis=-1).astype(np.float32)
    sin = np.repeat(np.sin(ang), 2, axis=-1).astype(np.float32)
    cos_t = np.tile(cos, (B, NH))
    sin_t = np.tile(sin, (B, NH))
    p = np.zeros((DH, DH), dtype=np.float32)
    for k in range(DH // 2):
        p[2 * k + 1, 2 * k] = -1.0
        p[2 * k, 2 * k + 1] = 1.0
    p4 = np.kron(np.eye(NH, dtype=np.float32), p)
    return cos_t, sin_t, p4


def kernel(x, Wq, Wk, Wv, Wo):
    cos_t, sin_t, p4 = _rope_tables()

    w_stacked = jnp.concatenate([Wq, Wk, Wv, Wo.T], axis=0).astype(BF16)
    x2 = x.reshape(B * SQ, D).astype(BF16)

    def body(x_ref, w_ref, p_ref, cos_ref, sin_ref, out_ref,
             gbuf_r, gbuf_l, ctx_buf,
             send_r, recv_r, send_l, recv_l):
        me = lax.axis_index("i")
        right = lax.rem(me + 1, N_DEV)
        left = lax.rem(me + N_DEV - 1, N_DEV)

        barrier_sem = pltpu.get_barrier_semaphore()
        for nbr in (left, right):
            pl.semaphore_signal(
                barrier_sem, inc=1,
                device_id=(nbr,), device_id_type=pl.DeviceIdType.MESH,
            )
        pl.semaphore_wait(barrier_sem, 2)

        xv = x_ref[...]
        cs = cos_ref[...]
        sn = sin_ref[...]
        pm = p_ref[...]

        def compute_chunk(blk, first=False):
            wq = blk[0 * D:1 * D]
            wk = blk[1 * D:2 * D]
            wv = blk[2 * D:3 * D]
            woT = blk[3 * D:4 * D]
            q = jnp.dot(xv, wq, preferred_element_type=F32)
            k = jnp.dot(xv, wk, preferred_element_type=F32)
            v = jnp.dot(xv, wv, preferred_element_type=F32)
            qr = q * cs + jnp.dot(q.astype(BF16), pm,
                                  preferred_element_type=F32) * sn
            kr = k * cs + jnp.dot(k.astype(BF16), pm,
                                  preferred_element_type=F32) * sn
            qr = qr.astype(BF16)
            kr = kr.astype(BF16)
            vb = v.astype(BF16)
            for b in range(B):
                rs = slice(b * SQ, (b + 1) * SQ)
                for hh in range(NH):
                    cs_h = slice(hh * DH, (hh + 1) * DH)
                    qh = qr[rs, cs_h]
                    kh = kr[rs, cs_h]
                    vh = vb[rs, cs_h]
                    sc = lax.dot_general(
                        qh, kh, (((1,), (1,)), ((), ())),
                        preferred_element_type=F32,
                    ) * 0.125
                    m = jnp.max(sc, axis=1, keepdims=True)
                    e = jnp.exp(sc - m)
                    sm = (e / jnp.sum(e, axis=1, keepdims=True)).astype(BF16)
                    ctx_buf[rs, cs_h] = jnp.dot(
                        sm, vh, preferred_element_type=F32).astype(BF16)
            contrib = lax.dot_general(
                ctx_buf[...], woT, (((1,), (1,)), ((), ())),
                preferred_element_type=F32,
            )
            if first:
                out_ref[...] = contrib
            else:
                out_ref[...] += contrib

        HF = 2 * D

        def make(tag, h, s):
            buf = gbuf_r if tag == "r" else gbuf_l
            ssem = send_r if tag == "r" else send_l
            rsem = recv_r if tag == "r" else recv_l
            dev = right if tag == "r" else left
            src = (w_ref.at[pl.ds(s * HF, HF)] if h == 1
                   else buf.at[h - 2, pl.ds(s * HF, HF)])
            return pltpu.make_async_remote_copy(
                src_ref=src,
                dst_ref=buf.at[h - 1, pl.ds(s * HF, HF)],
                send_sem=ssem.at[h - 1, s],
                recv_sem=rsem.at[h - 1, s],
                device_id=(dev,),
                device_id_type=pl.DeviceIdType.MESH,
            )

        descs = {}
        for tag in ("r", "l"):
            for s in range(2):
                d = make(tag, 1, s)
                d.start()
                descs[(tag, 1, s)] = d
        compute_chunk(w_ref[...], first=True)

        for h in range(2, NR + 1):
            for s in range(2):
                descs[("r", h - 1, s)].wait_recv()
                d = make("r", h, s)
                d.start()
                descs[("r", h, s)] = d
            if h <= NL:
                for s in range(2):
                    descs[("l", h - 1, s)].wait_recv()
                    d = make("l", h, s)
                    d.start()
                    descs[("l", h, s)] = d
                compute_chunk(gbuf_l[h - 2])
            compute_chunk(gbuf_r[h - 2])

        for s in range(2):
            descs[("l", NL, s)].wait_recv()
        compute_chunk(gbuf_l[NL - 1])
        for s in range(2):
            descs[("r", NR, s)].wait_recv()
        compute_chunk(gbuf_r[NR - 1])

        for d in descs.values():
            d.wait_send()

    out = pl.pallas_call(
        body,
        out_shape=jax.ShapeDtypeStruct((B * SQ, D), F32),
        in_specs=[
            pl.BlockSpec(memory_space=pltpu.VMEM),
            pl.BlockSpec(memory_space=pltpu.VMEM),
            pl.BlockSpec(memory_space=pltpu.VMEM),
            pl.BlockSpec(memory_space=pltpu.VMEM),
            pl.BlockSpec(memory_space=pltpu.VMEM),
        ],
        out_specs=pl.BlockSpec(memory_space=pltpu.VMEM),
        scratch_shapes=[
            pltpu.VMEM((NR, 4 * D, HL), BF16),
            pltpu.VMEM((NL, 4 * D, HL), BF16),
            pltpu.VMEM((B * SQ, HL), BF16),
            pltpu.SemaphoreType.DMA((NR, 2)),
            pltpu.SemaphoreType.DMA((NR, 2)),
            pltpu.SemaphoreType.DMA((NL, 2)),
            pltpu.SemaphoreType.DMA((NL, 2)),
        ],
        compiler_params=pltpu.CompilerParams(
            collective_id=0,
            vmem_limit_bytes=64 * 1024 * 1024,
        ),
    )(x2, w_stacked, jnp.asarray(p4, dtype=BF16),
      jnp.asarray(cos_t), jnp.asarray(sin_t))

    return out.reshape(B, SQ, D)
